# Initial kernel scaffold; baseline (speedup 1.0000x reference)
import jax
import jax.numpy as jnp
from jax.experimental import pallas as pl


def kernel(X, edge_index, Wz, bz, Lzw, Lzb, Wr, br, Lrw, Lrb, Wh, bh, Lhw, Lhb):
    raise NotImplementedError("write your pallas kernel here")


# speedup vs baseline: 150.3740x; 150.3740x over previous
import functools

import jax
import jax.numpy as jnp
from jax import lax
from jax.experimental import pallas as pl
from jax.experimental.pallas import tpu as pltpu
from jax.experimental.pallas import tpu_sc as plsc

B = 4
N = 10000
E = 160000
OUT = 64

NC = 2
NT = 16
L = 16
NP = 10240
SW = NP // NT
EC = E // NT
CH = 80
NCH = EC // CH


def _rsqrt(d):
    y = jnp.full(d.shape, 0.5, jnp.float32)
    for k in range(1, 9):
        y = jnp.where(d >= jnp.float32(4.0 ** k),
                      jnp.float32(2.0 ** (-(k + 1))), y)
    for _ in range(6):
        y = y * (1.5 - 0.5 * d * y * y)
    return y


def _sc_body(xp, ef, s_out,
             rowz, colz_f, colz, ones_v, vals_v, xs,
             dstr, ustr, ystr, tstr, sstr, zstr,
             u_loc, deg_sh, t_sh, u_sh):
    c = lax.axis_index("c")
    t = lax.axis_index("s")
    base = t * SW

    def _fill_ones(i, _):
        ones_v[pl.ds(i * L, L)] = jnp.full((L,), 1.0, jnp.float32)
        return 0
    lax.fori_loop(0, CH // L, _fill_ones, 0)

    def _fill_zero(i, _):
        zstr[pl.ds(i * L, L)] = jnp.zeros((L,), jnp.float32)
        return 0
    lax.fori_loop(0, SW // L, _fill_zero, 0)

    def _per_batch(p, _):
        b = c * 2 + p

        pltpu.sync_copy(zstr, deg_sh.at[pl.ds(base, SW)])
        pltpu.sync_copy(zstr, t_sh.at[pl.ds(base, SW)])
        pltpu.sync_copy(ef.at[pl.ds(b * 2 * E + t * EC, EC)], rowz)
        pltpu.sync_copy(ef.at[pl.ds((b * 2 + 1) * E + t * EC, EC)], colz_f)
        pltpu.sync_copy(xp.at[pl.ds(b * NP + base, SW)], xs)
        plsc.subcore_barrier()

        def _deg(j, _):
            for k in range(CH // L):
                colz[j, pl.ds(k * L, L)] = colz_f[pl.ds(j * CH + k * L, L)]
            pltpu.sync_copy(ones_v, deg_sh.at[colz.at[j]], add=True)
            return 0
        lax.fori_loop(0, NCH, _deg, 0)
        plsc.subcore_barrier()

        pltpu.sync_copy(deg_sh.at[pl.ds(base, SW)], dstr)

        def _du(i, _):
            sl = pl.ds(i * L, L)
            d = dstr[sl] + 1.0
            y = _rsqrt(d)
            ystr[sl] = y
            ustr[sl] = y * xs[sl]
            return 0
        lax.fori_loop(0, SW // L, _du, 0)
        pltpu.sync_copy(ustr, u_sh.at[pl.ds(base, SW)])
        plsc.subcore_barrier()

        pltpu.sync_copy(u_sh, u_loc)

        def _gs(j, _):
            for k in range(CH // L):
                sl = pl.ds(j * CH + k * L, L)
                rv = rowz[sl]
                vals_v[sl] = plsc.load_gather(u_loc, [rv])
            pltpu.sync_copy(vals_v.at[pl.ds(j * CH, CH)],
                            t_sh.at[colz.at[j]], add=True)
            return 0
        lax.fori_loop(0, NCH, _gs, 0)
        plsc.subcore_barrier()

        pltpu.sync_copy(t_sh.at[pl.ds(base, SW)], tstr)

        def _s(i, _):
            sl = pl.ds(i * L, L)
            sstr[sl] = ystr[sl] * (tstr[sl] + ustr[sl])
            return 0
        lax.fori_loop(0, SW // L, _s, 0)
        pltpu.sync_copy(sstr, s_out.at[pl.ds(b * NP + base, SW)])
        return 0

    lax.fori_loop(0, B // NC, _per_batch, 0)


def _sc_segment_s(xp, ef):
    mesh = plsc.VectorSubcoreMesh(core_axis_name="c", subcore_axis_name="s")
    f = pl.kernel(
        _sc_body,
        out_type=jax.ShapeDtypeStruct((B * NP,), jnp.float32),
        mesh=mesh,
        compiler_params=pltpu.CompilerParams(needs_layout_passes=False),
        scratch_types=[
            pltpu.VMEM((EC,), jnp.int32),
            pltpu.VMEM((EC,), jnp.int32),
            pltpu.VMEM((NCH, CH), jnp.int32),
            pltpu.VMEM((CH,), jnp.float32),
            pltpu.VMEM((EC,), jnp.float32),
            pltpu.VMEM((SW,), jnp.float32),
            pltpu.VMEM((SW,), jnp.float32),
            pltpu.VMEM((SW,), jnp.float32),
            pltpu.VMEM((SW,), jnp.float32),
            pltpu.VMEM((SW,), jnp.float32),
            pltpu.VMEM((SW,), jnp.float32),
            pltpu.VMEM((SW,), jnp.float32),
            pltpu.VMEM((NP,), jnp.float32),
            pltpu.VMEM_SHARED((NP,), jnp.float32),
            pltpu.VMEM_SHARED((NP,), jnp.float32),
            pltpu.VMEM_SHARED((NP,), jnp.float32),
        ],
    )
    return f(xp, ef)


BN = 2560


def _tc_body(s_ref, wz, bz, lzw, lzb, wh, bh, lhw, lhb, o_ref):
    az = jnp.dot(wz[...], lzw[:OUT, :])
    cz = jnp.dot(bz[...], lzw[:OUT, :]) + lzb[...]
    ah = jnp.dot(wh[...], lhw[:OUT, :])
    ch = jnp.dot(bh[...], lhw[:OUT, :]) + lhb[...]
    s = s_ref[...][:, :, None]
    z = jax.nn.sigmoid(s * az[0] + cz[0])
    ht = jnp.tanh(s * ah[0] + ch[0])
    o_ref[...] = (1.0 - z) * ht


def _tc_expand(s_pad, wz, bz, lzw, lzb, wh, bh, lhw, lhb):
    wspec = lambda shape: pl.BlockSpec(shape, lambda j: (0,) * len(shape))
    return pl.pallas_call(
        _tc_body,
        grid=(NP // BN,),
        in_specs=[
            pl.BlockSpec((B, BN), lambda j: (0, j)),
            wspec((1, OUT)), wspec((1, OUT)), wspec((2 * OUT, OUT)),
            wspec((1, OUT)),
            wspec((1, OUT)), wspec((1, OUT)), wspec((2 * OUT, OUT)),
            wspec((1, OUT)),
        ],
        out_specs=pl.BlockSpec((B, BN, OUT), lambda j: (0, j, 0)),
        out_shape=jax.ShapeDtypeStruct((B, NP, OUT), jnp.float32),
    )(s_pad, wz, bz, lzw, lzb, wh, bh, lhw, lhb)


@jax.jit
def kernel(X, edge_index, Wz, bz, Lzw, Lzb, Wr, br, Lrw, Lrb, Wh, bh, Lhw, Lhb):
    del Wr, br, Lrw, Lrb
    xp = jnp.pad(X.reshape(B, N), ((0, 0), (0, NP - N)))
    m = jnp.min(edge_index, axis=(1, 2), keepdims=True)
    ef = (edge_index - m).astype(jnp.int32)
    s_pad = _sc_segment_s(xp.reshape(-1), ef.reshape(-1)).reshape(B, NP)
    out = _tc_expand(
        s_pad,
        Wz.reshape(1, OUT), bz.reshape(1, OUT), Lzw, Lzb.reshape(1, OUT),
        Wh.reshape(1, OUT), bh.reshape(1, OUT), Lhw, Lhb.reshape(1, OUT),
    )
    return out[:, :N, :]


# speedup vs baseline: 206.8051x; 1.3753x over previous
import functools

import jax
import jax.numpy as jnp
from jax import lax
from jax.experimental import pallas as pl
from jax.experimental.pallas import tpu as pltpu
from jax.experimental.pallas import tpu_sc as plsc

B = 4
N = 10000
E = 160000
OUT = 64

NC = 2
NT = 16
L = 16
NP = 10240
SW = NP // NT
EC = E // NT
CH = 80
NCH = EC // CH


def _rsqrt(d):
    y = jnp.full(d.shape, 0.5, jnp.float32)
    for k in range(1, 9):
        y = jnp.where(d >= jnp.float32(4.0 ** k),
                      jnp.float32(2.0 ** (-(k + 1))), y)
    for _ in range(6):
        y = y * (1.5 - 0.5 * d * y * y)
    return y


def _sc_body(xp, ef, s_out,
             rowz, colz_f, colz, ones_v, vals_v, xs,
             dstr, ustr, ystr, tstr, sstr, zstr,
             u_loc, deg_sh, t_sh, u_sh, sem):
    c = lax.axis_index("c")
    t = lax.axis_index("s")
    base = t * SW

    def _fill_ones(i, _):
        ones_v[pl.ds(i * L, L)] = jnp.full((L,), 1.0, jnp.float32)
        return 0
    lax.fori_loop(0, CH // L, _fill_ones, 0)

    def _fill_zero(i, _):
        zstr[pl.ds(i * L, L)] = jnp.zeros((L,), jnp.float32)
        return 0
    lax.fori_loop(0, SW // L, _fill_zero, 0)

    def _per_batch(p, _):
        b = c * 2 + p

        pltpu.sync_copy(zstr, deg_sh.at[pl.ds(base, SW)])
        pltpu.sync_copy(zstr, t_sh.at[pl.ds(base, SW)])
        pltpu.sync_copy(ef.at[pl.ds(b * 2 * E + t * EC, EC)], rowz)
        pltpu.sync_copy(ef.at[pl.ds((b * 2 + 1) * E + t * EC, EC)], colz_f)
        pltpu.sync_copy(xp.at[pl.ds(b * NP + base, SW)], xs)
        plsc.subcore_barrier()

        def _deg(j, _):
            for k in range(CH // L):
                colz[j, pl.ds(k * L, L)] = colz_f[pl.ds(j * CH + k * L, L)]
            pltpu.async_copy(ones_v, deg_sh.at[colz.at[j]], sem, add=True)
            return 0
        lax.fori_loop(0, NCH, _deg, 0)
        pltpu.make_async_copy(xp.at[pl.ds(0, EC)], vals_v, sem).wait()
        plsc.subcore_barrier()

        pltpu.sync_copy(deg_sh.at[pl.ds(base, SW)], dstr)

        def _du(i, _):
            sl = pl.ds(i * L, L)
            d = dstr[sl] + 1.0
            y = _rsqrt(d)
            ystr[sl] = y
            ustr[sl] = y * xs[sl]
            return 0
        lax.fori_loop(0, SW // L, _du, 0)
        pltpu.sync_copy(ustr, u_sh.at[pl.ds(base, SW)])
        plsc.subcore_barrier()

        pltpu.sync_copy(u_sh, u_loc)

        def _gs(j, _):
            for k in range(CH // L):
                sl = pl.ds(j * CH + k * L, L)
                rv = rowz[sl]
                vals_v[sl] = plsc.load_gather(u_loc, [rv])
            pltpu.async_copy(vals_v.at[pl.ds(j * CH, CH)],
                             t_sh.at[colz.at[j]], sem, add=True)
            return 0
        lax.fori_loop(0, NCH, _gs, 0)
        pltpu.make_async_copy(xp.at[pl.ds(0, EC)], vals_v, sem).wait()
        plsc.subcore_barrier()

        pltpu.sync_copy(t_sh.at[pl.ds(base, SW)], tstr)

        def _s(i, _):
            sl = pl.ds(i * L, L)
            sstr[sl] = ystr[sl] * (tstr[sl] + ustr[sl])
            return 0
        lax.fori_loop(0, SW // L, _s, 0)
        pltpu.sync_copy(sstr, s_out.at[pl.ds(b * NP + base, SW)])
        return 0

    lax.fori_loop(0, B // NC, _per_batch, 0)


def _sc_segment_s(xp, ef):
    mesh = plsc.VectorSubcoreMesh(core_axis_name="c", subcore_axis_name="s")
    f = pl.kernel(
        _sc_body,
        out_type=jax.ShapeDtypeStruct((B * NP,), jnp.float32),
        mesh=mesh,
        compiler_params=pltpu.CompilerParams(needs_layout_passes=False),
        scratch_types=[
            pltpu.VMEM((EC,), jnp.int32),
            pltpu.VMEM((EC,), jnp.int32),
            pltpu.VMEM((NCH, CH), jnp.int32),
            pltpu.VMEM((CH,), jnp.float32),
            pltpu.VMEM((EC,), jnp.float32),
            pltpu.VMEM((SW,), jnp.float32),
            pltpu.VMEM((SW,), jnp.float32),
            pltpu.VMEM((SW,), jnp.float32),
            pltpu.VMEM((SW,), jnp.float32),
            pltpu.VMEM((SW,), jnp.float32),
            pltpu.VMEM((SW,), jnp.float32),
            pltpu.VMEM((SW,), jnp.float32),
            pltpu.VMEM((NP,), jnp.float32),
            pltpu.VMEM_SHARED((NP,), jnp.float32),
            pltpu.VMEM_SHARED((NP,), jnp.float32),
            pltpu.VMEM_SHARED((NP,), jnp.float32),
            pltpu.SemaphoreType.DMA,
        ],
    )
    return f(xp, ef)


BN = 2000


def _tc_body(s_ref, wz, bz, lzw, lzb, wh, bh, lhw, lhb, o_ref):
    az = jnp.dot(wz[...], lzw[:OUT, :])
    cz = jnp.dot(bz[...], lzw[:OUT, :]) + lzb[...]
    ah = jnp.dot(wh[...], lhw[:OUT, :])
    ch = jnp.dot(bh[...], lhw[:OUT, :]) + lhb[...]
    s = s_ref[...]
    z = jax.nn.sigmoid(s * az[0] + cz[0])
    ht = jnp.tanh(s * ah[0] + ch[0])
    o_ref[...] = (1.0 - z) * ht


def _tc_expand(s3, wz, bz, lzw, lzb, wh, bh, lhw, lhb):
    wspec = lambda shape: pl.BlockSpec(shape, lambda j: (0,) * len(shape))
    return pl.pallas_call(
        _tc_body,
        grid=(N // BN,),
        in_specs=[
            pl.BlockSpec((B, BN, 1), lambda j: (0, j, 0)),
            wspec((1, OUT)), wspec((1, OUT)), wspec((2 * OUT, OUT)),
            wspec((1, OUT)),
            wspec((1, OUT)), wspec((1, OUT)), wspec((2 * OUT, OUT)),
            wspec((1, OUT)),
        ],
        out_specs=pl.BlockSpec((B, BN, OUT), lambda j: (0, j, 0)),
        out_shape=jax.ShapeDtypeStruct((B, N, OUT), jnp.float32),
    )(s3, wz, bz, lzw, lzb, wh, bh, lhw, lhb)


@jax.jit
def kernel(X, edge_index, Wz, bz, Lzw, Lzb, Wr, br, Lrw, Lrb, Wh, bh, Lhw, Lhb):
    del Wr, br, Lrw, Lrb
    xp = jnp.pad(X.reshape(B, N), ((0, 0), (0, NP - N)))
    m = jnp.min(edge_index, axis=(1, 2), keepdims=True)
    ef = (edge_index - m).astype(jnp.int32)
    s_pad = _sc_segment_s(xp.reshape(-1), ef.reshape(-1)).reshape(B, NP)
    s3 = s_pad[:, :N, None]
    return _tc_expand(
        s3,
        Wz.reshape(1, OUT), bz.reshape(1, OUT), Lzw, Lzb.reshape(1, OUT),
        Wh.reshape(1, OUT), bh.reshape(1, OUT), Lhw, Lhb.reshape(1, OUT),
    )


# speedup vs baseline: 218.0373x; 1.0543x over previous
import functools

import jax
import jax.numpy as jnp
from jax import lax
from jax.experimental import pallas as pl
from jax.experimental.pallas import tpu as pltpu
from jax.experimental.pallas import tpu_sc as plsc

B = 4
N = 10000
E = 160000
OUT = 64

NC = 2
NT = 16
L = 16
NP = 10240
SW = NP // NT
EC = E // NT
CH = 80
NCH = EC // CH


def _rsqrt(d):
    y = jnp.full(d.shape, 0.5, jnp.float32)
    for k in range(1, 9):
        y = jnp.where(d >= jnp.float32(4.0 ** k),
                      jnp.float32(2.0 ** (-(k + 1))), y)
    for _ in range(6):
        y = y * (1.5 - 0.5 * d * y * y)
    return y


def _sc_body(xp, ef, s_out,
             rowz, colz_f, colz, ones_v, vals_v, xs,
             dstr, ustr, ystr, tstr, sstr, zstr,
             u_loc, deg_sh, t_sh, u_sh, sem):
    c = lax.axis_index("c")
    t = lax.axis_index("s")
    base = t * SW

    def _fill_ones(i, _):
        ones_v[pl.ds(i * L, L)] = jnp.full((L,), 1.0, jnp.float32)
        return 0
    lax.fori_loop(0, CH // L, _fill_ones, 0)

    def _fill_zero(i, _):
        zstr[pl.ds(i * L, L)] = jnp.zeros((L,), jnp.float32)
        return 0
    lax.fori_loop(0, SW // L, _fill_zero, 0)

    def _per_batch(p, _):
        b = c * 2 + p

        pltpu.sync_copy(zstr, deg_sh.at[pl.ds(base, SW)])
        pltpu.sync_copy(zstr, t_sh.at[pl.ds(base, SW)])
        pltpu.sync_copy(ef.at[pl.ds(b * 2 * E + t * EC, EC)], rowz)
        pltpu.sync_copy(ef.at[pl.ds((b * 2 + 1) * E + t * EC, EC)], colz_f)
        pltpu.sync_copy(xp.at[pl.ds(b * NP + base, SW)], xs)
        plsc.subcore_barrier()

        def _deg(j, _):
            for k in range(CH // L):
                colz[j, pl.ds(k * L, L)] = colz_f[pl.ds(j * CH + k * L, L)]
            pltpu.async_copy(ones_v, deg_sh.at[colz.at[j]], sem, add=True)
            return 0
        lax.fori_loop(0, NCH, _deg, 0)
        pltpu.make_async_copy(xp.at[pl.ds(0, EC)], vals_v, sem).wait()
        plsc.subcore_barrier()

        pltpu.sync_copy(deg_sh.at[pl.ds(base, SW)], dstr)

        def _du(i, _):
            sl = pl.ds(i * L, L)
            d = dstr[sl] + 1.0
            y = _rsqrt(d)
            ystr[sl] = y
            ustr[sl] = y * xs[sl]
            return 0
        lax.fori_loop(0, SW // L, _du, 0)
        pltpu.sync_copy(ustr, u_sh.at[pl.ds(base, SW)])
        plsc.subcore_barrier()

        pltpu.sync_copy(u_sh, u_loc)

        def _gs(j, _):
            for k in range(CH // L):
                sl = pl.ds(j * CH + k * L, L)
                rv = rowz[sl]
                vals_v[sl] = plsc.load_gather(u_loc, [rv])
            pltpu.async_copy(vals_v.at[pl.ds(j * CH, CH)],
                             t_sh.at[colz.at[j]], sem, add=True)
            return 0
        lax.fori_loop(0, NCH, _gs, 0)
        pltpu.make_async_copy(xp.at[pl.ds(0, EC)], vals_v, sem).wait()
        plsc.subcore_barrier()

        pltpu.sync_copy(t_sh.at[pl.ds(base, SW)], tstr)

        def _s(i, _):
            sl = pl.ds(i * L, L)
            sstr[sl] = ystr[sl] * (tstr[sl] + ustr[sl])
            return 0
        lax.fori_loop(0, SW // L, _s, 0)
        pltpu.sync_copy(sstr, s_out.at[pl.ds(b * NP + base, SW)])
        return 0

    lax.fori_loop(0, B // NC, _per_batch, 0)


def _sc_segment_s(xp, ef):
    mesh = plsc.VectorSubcoreMesh(core_axis_name="c", subcore_axis_name="s")
    f = pl.kernel(
        _sc_body,
        out_type=jax.ShapeDtypeStruct((B * NP,), jnp.float32),
        mesh=mesh,
        compiler_params=pltpu.CompilerParams(needs_layout_passes=False),
        scratch_types=[
            pltpu.VMEM((EC,), jnp.int32),
            pltpu.VMEM((EC,), jnp.int32),
            pltpu.VMEM((NCH, CH), jnp.int32),
            pltpu.VMEM((CH,), jnp.float32),
            pltpu.VMEM((EC,), jnp.float32),
            pltpu.VMEM((SW,), jnp.float32),
            pltpu.VMEM((SW,), jnp.float32),
            pltpu.VMEM((SW,), jnp.float32),
            pltpu.VMEM((SW,), jnp.float32),
            pltpu.VMEM((SW,), jnp.float32),
            pltpu.VMEM((SW,), jnp.float32),
            pltpu.VMEM((SW,), jnp.float32),
            pltpu.VMEM((NP,), jnp.float32),
            pltpu.VMEM_SHARED((NP,), jnp.float32),
            pltpu.VMEM_SHARED((NP,), jnp.float32),
            pltpu.VMEM_SHARED((NP,), jnp.float32),
            pltpu.SemaphoreType.DMA,
        ],
    )
    return f(xp, ef)


BN = 2048


def _tc_body(s_ref, wz, bz, lzw, lzb, wh, bh, lhw, lhb, o_ref):
    az = jnp.dot(wz[...], lzw[:OUT, :])
    cz = jnp.dot(bz[...], lzw[:OUT, :]) + lzb[...]
    ah = jnp.dot(wh[...], lhw[:OUT, :])
    ch = jnp.dot(bh[...], lhw[:OUT, :]) + lhb[...]
    s = s_ref[...][:, :, None]
    z = jax.nn.sigmoid(s * az[0] + cz[0])
    ht = jnp.tanh(s * ah[0] + ch[0])
    o_ref[...] = (1.0 - z) * ht


def _tc_expand(s_pad, wz, bz, lzw, lzb, wh, bh, lhw, lhb):
    wspec = lambda shape: pl.BlockSpec(shape, lambda j: (0,) * len(shape))
    return pl.pallas_call(
        _tc_body,
        grid=(pl.cdiv(N, BN),),
        in_specs=[
            pl.BlockSpec((B, BN), lambda j: (0, j)),
            wspec((1, OUT)), wspec((1, OUT)), wspec((2 * OUT, OUT)),
            wspec((1, OUT)),
            wspec((1, OUT)), wspec((1, OUT)), wspec((2 * OUT, OUT)),
            wspec((1, OUT)),
        ],
        out_specs=pl.BlockSpec((B, BN, OUT), lambda j: (0, j, 0)),
        out_shape=jax.ShapeDtypeStruct((B, N, OUT), jnp.float32),
    )(s_pad, wz, bz, lzw, lzb, wh, bh, lhw, lhb)


@jax.jit
def kernel(X, edge_index, Wz, bz, Lzw, Lzb, Wr, br, Lrw, Lrb, Wh, bh, Lhw, Lhb):
    del Wr, br, Lrw, Lrb
    xp = jnp.pad(X.reshape(B, N), ((0, 0), (0, NP - N)))
    ei_flat = edge_index.reshape(B, 2 * E)
    m = jnp.min(ei_flat, axis=1, keepdims=True)
    ef = (ei_flat - m).astype(jnp.int32)
    s_pad = _sc_segment_s(xp.reshape(-1), ef.reshape(-1)).reshape(B, NP)
    return _tc_expand(
        s_pad,
        Wz.reshape(1, OUT), bz.reshape(1, OUT), Lzw, Lzb.reshape(1, OUT),
        Wh.reshape(1, OUT), bh.reshape(1, OUT), Lhw, Lhb.reshape(1, OUT),
    )


# speedup vs baseline: 248.0092x; 1.1375x over previous
import functools

import jax
import jax.numpy as jnp
from jax import lax
from jax.experimental import pallas as pl
from jax.experimental.pallas import tpu as pltpu
from jax.experimental.pallas import tpu_sc as plsc

B = 4
N = 10000
E = 160000
OUT = 64

NC = 2
NT = 16
L = 16
NP = 10240
SW = NP // NT
EC = E // NT
CH = 80
NCH = EC // CH


def _rsqrt(d):
    y = jnp.full(d.shape, 0.5, jnp.float32)
    for k in range(1, 9):
        y = jnp.where(d >= jnp.float32(4.0 ** k),
                      jnp.float32(2.0 ** (-(k + 1))), y)
    for _ in range(6):
        y = y * (1.5 - 0.5 * d * y * y)
    return y


def _sc_body(xp, ef, s_out,
             rowz, colz_f, colz, ones_v, vals_v, xs,
             dstr, ustr, ystr, tstr, sstr, zstr,
             minb, mall, u_loc, min_sh, deg_sh, t_sh, u_sh, sem):
    c = lax.axis_index("c")
    t = lax.axis_index("s")
    base = t * SW

    def _fill_ones(i, _):
        ones_v[pl.ds(i * L, L)] = jnp.full((L,), 1.0, jnp.float32)
        return 0
    lax.fori_loop(0, CH // L, _fill_ones, 0)

    def _fill_zero(i, _):
        zstr[pl.ds(i * L, L)] = jnp.zeros((L,), jnp.float32)
        return 0
    lax.fori_loop(0, SW // L, _fill_zero, 0)

    def _per_batch(p, _):
        b = c * 2 + p

        pltpu.sync_copy(zstr, deg_sh.at[pl.ds(base, SW)])
        pltpu.sync_copy(zstr, t_sh.at[pl.ds(base, SW)])
        pltpu.sync_copy(ef.at[pl.ds(b * 2 * E + t * EC, EC)], rowz)
        pltpu.sync_copy(ef.at[pl.ds((b * 2 + 1) * E + t * EC, EC)], colz_f)
        pltpu.sync_copy(xp.at[pl.ds(b * NP + base, SW)], xs)

        def _lmin(i, acc):
            sl = pl.ds(i * L, L)
            return jnp.minimum(acc, jnp.minimum(rowz[sl], colz_f[sl]))
        macc = lax.fori_loop(0, EC // L, _lmin,
                             jnp.full((L,), jnp.int32(2147483647)))
        minb[...] = macc
        pltpu.sync_copy(minb, min_sh.at[t])
        plsc.subcore_barrier()
        pltpu.sync_copy(min_sh, mall)
        gacc = mall[0]
        for i in range(1, NT):
            gacc = jnp.minimum(gacc, mall[i])
        m = jnp.min(gacc, axis=0)

        def _deg(j, _):
            for k in range(CH // L):
                colz[j, pl.ds(k * L, L)] = colz_f[pl.ds(j * CH + k * L, L)] - m
            pltpu.async_copy(ones_v, deg_sh.at[colz.at[j]], sem, add=True)
            return 0
        lax.fori_loop(0, NCH, _deg, 0)
        pltpu.make_async_copy(xp.at[pl.ds(0, EC)], vals_v, sem).wait()
        plsc.subcore_barrier()

        pltpu.sync_copy(deg_sh.at[pl.ds(base, SW)], dstr)

        def _du(i, _):
            sl = pl.ds(i * L, L)
            d = dstr[sl] + 1.0
            y = _rsqrt(d)
            ystr[sl] = y
            ustr[sl] = y * xs[sl]
            return 0
        lax.fori_loop(0, SW // L, _du, 0)
        pltpu.sync_copy(ustr, u_sh.at[pl.ds(base, SW)])
        plsc.subcore_barrier()

        pltpu.sync_copy(u_sh, u_loc)

        def _gs(j, _):
            for k in range(CH // L):
                sl = pl.ds(j * CH + k * L, L)
                rv = rowz[sl] - m
                vals_v[sl] = plsc.load_gather(u_loc, [rv])
            pltpu.async_copy(vals_v.at[pl.ds(j * CH, CH)],
                             t_sh.at[colz.at[j]], sem, add=True)
            return 0
        lax.fori_loop(0, NCH, _gs, 0)
        pltpu.make_async_copy(xp.at[pl.ds(0, EC)], vals_v, sem).wait()
        plsc.subcore_barrier()

        pltpu.sync_copy(t_sh.at[pl.ds(base, SW)], tstr)

        def _s(i, _):
            sl = pl.ds(i * L, L)
            sstr[sl] = ystr[sl] * (tstr[sl] + ustr[sl])
            return 0
        lax.fori_loop(0, SW // L, _s, 0)
        pltpu.sync_copy(sstr, s_out.at[pl.ds(b * NP + base, SW)])
        return 0

    lax.fori_loop(0, B // NC, _per_batch, 0)


def _sc_segment_s(xp, ef):
    mesh = plsc.VectorSubcoreMesh(core_axis_name="c", subcore_axis_name="s")
    f = pl.kernel(
        _sc_body,
        out_type=jax.ShapeDtypeStruct((B * NP,), jnp.float32),
        mesh=mesh,
        compiler_params=pltpu.CompilerParams(needs_layout_passes=False),
        scratch_types=[
            pltpu.VMEM((EC,), jnp.int32),
            pltpu.VMEM((EC,), jnp.int32),
            pltpu.VMEM((NCH, CH), jnp.int32),
            pltpu.VMEM((CH,), jnp.float32),
            pltpu.VMEM((EC,), jnp.float32),
            pltpu.VMEM((SW,), jnp.float32),
            pltpu.VMEM((SW,), jnp.float32),
            pltpu.VMEM((SW,), jnp.float32),
            pltpu.VMEM((SW,), jnp.float32),
            pltpu.VMEM((SW,), jnp.float32),
            pltpu.VMEM((SW,), jnp.float32),
            pltpu.VMEM((SW,), jnp.float32),
            pltpu.VMEM((L,), jnp.int32),
            pltpu.VMEM((NT, L), jnp.int32),
            pltpu.VMEM((NP,), jnp.float32),
            pltpu.VMEM_SHARED((NT, L), jnp.int32),
            pltpu.VMEM_SHARED((NP,), jnp.float32),
            pltpu.VMEM_SHARED((NP,), jnp.float32),
            pltpu.VMEM_SHARED((NP,), jnp.float32),
            pltpu.SemaphoreType.DMA,
        ],
    )
    return f(xp, ef)


BN = 2048


def _tc_body(s_ref, wz, bz, lzw, lzb, wh, bh, lhw, lhb, o_ref):
    az = jnp.dot(wz[...], lzw[:OUT, :])
    cz = jnp.dot(bz[...], lzw[:OUT, :]) + lzb[...]
    ah = jnp.dot(wh[...], lhw[:OUT, :])
    ch = jnp.dot(bh[...], lhw[:OUT, :]) + lhb[...]
    s = s_ref[...][:, :, None]
    z = jax.nn.sigmoid(s * az[0] + cz[0])
    ht = jnp.tanh(s * ah[0] + ch[0])
    o_ref[...] = (1.0 - z) * ht


def _tc_expand(s_pad, wz, bz, lzw, lzb, wh, bh, lhw, lhb):
    wspec = lambda shape: pl.BlockSpec(shape, lambda j: (0,) * len(shape))
    return pl.pallas_call(
        _tc_body,
        grid=(pl.cdiv(N, BN),),
        in_specs=[
            pl.BlockSpec((B, BN), lambda j: (0, j)),
            wspec((1, OUT)), wspec((1, OUT)), wspec((2 * OUT, OUT)),
            wspec((1, OUT)),
            wspec((1, OUT)), wspec((1, OUT)), wspec((2 * OUT, OUT)),
            wspec((1, OUT)),
        ],
        out_specs=pl.BlockSpec((B, BN, OUT), lambda j: (0, j, 0)),
        out_shape=jax.ShapeDtypeStruct((B, N, OUT), jnp.float32),
    )(s_pad, wz, bz, lzw, lzb, wh, bh, lhw, lhb)


@jax.jit
def kernel(X, edge_index, Wz, bz, Lzw, Lzb, Wr, br, Lrw, Lrb, Wh, bh, Lhw, Lhb):
    del Wr, br, Lrw, Lrb
    xp = jnp.pad(X.reshape(B, N), ((0, 0), (0, NP - N)))
    ef = edge_index.astype(jnp.int32).reshape(-1)
    s_pad = _sc_segment_s(xp.reshape(-1), ef).reshape(B, NP)
    return _tc_expand(
        s_pad,
        Wz.reshape(1, OUT), bz.reshape(1, OUT), Lzw, Lzb.reshape(1, OUT),
        Wh.reshape(1, OUT), bh.reshape(1, OUT), Lhw, Lhb.reshape(1, OUT),
    )
